# baseline (speedup 1.0000x reference)
import functools

import jax
import jax.numpy as jnp
from jax.experimental import pallas as pl
from jax.experimental.pallas import tpu as pltpu

N = 16384
V = 100000
VP = 100352
D = 32
K = 8
H = 64

QB = 32
VT = 128
NT = VP // VT

BIG = 3.0e38
PAD_COORD = 1.0e4


def _knn_body(x_ref, vt_ref, idx_ref, d2_ref):
    x0 = x_ref[:, 0:1]
    x1 = x_ref[:, 1:2]
    x2 = x_ref[:, 2:3]
    xsq = x_ref[:, 3:4]

    def tile(t, _):
        off = pl.multiple_of(t * VT, VT)
        v = vt_ref[:, pl.ds(off, VT)]
        dot = (x0 * v[0:1, :] + x1 * v[1:2, :]) + x2 * v[2:3, :]
        d2_ref[:, pl.ds(off, VT)] = (xsq - 2.0 * dot) + v[3:4, :]
        return 0

    jax.lax.fori_loop(0, NT, tile, 0)

    ii = jax.lax.broadcasted_iota(jnp.int32, (QB, VP), 1)
    for k in range(K):
        d = d2_ref[:, :]
        m = jnp.min(d, axis=1, keepdims=True)
        sel = d == m
        am = jnp.min(jnp.where(sel, ii, jnp.int32(VP)), axis=1, keepdims=True)
        idx_ref[:, k] = am[:, 0]
        d2_ref[:, :] = jnp.where(ii == am, BIG, d)


def _round_bf16(x):
    u = jax.lax.bitcast_convert_type(x, jnp.uint32)
    u = (u + jnp.uint32(0x7FFF) + ((u >> 16) & jnp.uint32(1))) & jnp.uint32(0xFFFF0000)
    return jax.lax.bitcast_convert_type(u, jnp.float32)


def _sqnorm(a):
    return (a[:, 0] * a[:, 0] + a[:, 2] * a[:, 2]) + a[:, 1] * a[:, 1]


@functools.partial(jax.jit, static_argnums=())
def _knn(xyz, mesh_vertices):
    vpad = jnp.pad(mesh_vertices, ((0, VP - V), (0, 0)),
                   constant_values=PAD_COORD)
    vt = jnp.concatenate([_round_bf16(vpad), _sqnorm(vpad)[:, None]],
                         axis=1).T
    xq = jnp.concatenate([_round_bf16(xyz), _sqnorm(xyz)[:, None]],
                         axis=1)
    idx = pl.pallas_call(
        _knn_body,
        grid=(N // QB,),
        in_specs=[
            pl.BlockSpec((QB, 4), lambda i: (i, 0)),
            pl.BlockSpec((4, VP), lambda i: (0, 0)),
        ],
        out_specs=pl.BlockSpec((QB, K), lambda i: (i, 0)),
        out_shape=jax.ShapeDtypeStruct((N, K), jnp.int32),
        scratch_shapes=[pltpu.VMEM((QB, VP), jnp.float32)],
    )(xq, vt)
    return idx


def kernel(xyz, view_dirs, mesh_vertices, color_features, edit_color_features,
           geo_features, main_mask, W1, b1, W2, b2, Ws1, bs1, Ws2, bs2, Wg, bg):
    idx = _knn(xyz, mesh_vertices)

    neigh = mesh_vertices[idx]
    diff = xyz[:, None, :] - neigh
    ds = jnp.linalg.norm(diff, axis=-1)
    w = 1.0 / (ds + 1e-8)
    w = w / jnp.sum(w, axis=-1, keepdims=True)
    nabla = jnp.sum(w[..., None] * diff, axis=-2)
    nabla = nabla / (jnp.linalg.norm(nabla, axis=-1, keepdims=True) + 1e-8)
    feat = jnp.sum(w[..., None] * color_features[idx], axis=-2)
    geo = jnp.sum(w[..., None] * geo_features[idx], axis=-2)
    sdf = (geo @ Wg + bg).squeeze(-1)
    h = jax.nn.relu(jnp.concatenate([feat, view_dirs, nabla], axis=-1) @ W1 + b1)
    colors = jax.nn.sigmoid(h @ W2 + b2)
    mg = main_mask[idx]
    paint_region = jnp.sum(mg.astype(jnp.int32), axis=-1) >= K
    sw = w * mg.astype(w.dtype)
    sw = sw / (jnp.sum(sw, axis=-1, keepdims=True) + 1e-8)
    sfeat = jnp.sum(sw[..., None] * edit_color_features[idx], axis=-2)
    hs = jax.nn.relu(jnp.concatenate([sfeat, view_dirs, nabla], axis=-1) @ Ws1 + bs1)
    slave_color = jax.nn.sigmoid(hs @ Ws2 + bs2)
    blend_color = jnp.where(paint_region[:, None], slave_color, colors)
    return sdf, blend_color
